# baseline (speedup 1.0000x reference)
import functools

import jax
import jax.numpy as jnp
from jax import lax
from jax.experimental import pallas as pl
from jax.experimental.pallas import tpu as pltpu
from jax.experimental.pallas import tpu_sc as plsc

N_NODES = 50000
N_EDGES = 800000
V_IN = 16
H_DIM = 64
C_OUT = 6

UN = 128
KS = 5
KS2 = 5
SUP1 = 3
SUP2 = 6
E_ROWS = N_EDGES // UN
N_ACC = 51200
ROWS_PER_TILE = N_ACC // 16


def _sc1_body(x_hbm, ei_hbm, z16_hbm, ones_hbm,
              aggp_hbm, cntp_hbm,
              idx_v, rows_v, ones_v, acc_s, cnt_s, gsems, ssem):
    c = lax.axis_index("c")
    s = lax.axis_index("s")
    zb = s * ROWS_PER_TILE
    pltpu.sync_copy(z16_hbm.at[pl.ds(zb, ROWS_PER_TILE)],
                    acc_s.at[pl.ds(zb, ROWS_PER_TILE)])
    pltpu.sync_copy(z16_hbm.at[pl.ds(zb, ROWS_PER_TILE)],
                    cnt_s.at[pl.ds(zb, ROWS_PER_TILE)])
    pltpu.sync_copy(ones_hbm, ones_v)
    plsc.subcore_barrier()

    def group(q):
        gd = [pltpu.async_copy(x_hbm.at[idx_v.at[0, q * KS + j]],
                               rows_v.at[j], gsems.at[j])
              for j in range(KS)]
        sd = []
        for j in range(KS):
            gd[j].wait()
            sd.append(pltpu.async_copy(rows_v.at[j],
                                       acc_s.at[idx_v.at[1, q * KS + j]],
                                       ssem, add=True))
            sd.append(pltpu.async_copy(ones_v,
                                       cnt_s.at[idx_v.at[1, q * KS + j]],
                                       ssem, add=True))
        for d in sd:
            d.wait()

    rows_per_sc = E_ROWS // 2
    tile_rows = 195
    base = c * rows_per_sc + s * tile_rows
    sup_rows = KS * SUP1

    def body(g, _):
        r0 = base + g * sup_rows
        pltpu.sync_copy(ei_hbm.at[:, pl.ds(r0, sup_rows)], idx_v)
        for q in range(SUP1):
            group(q)
        return 0

    lax.fori_loop(0, tile_rows // sup_rows, body, 0)

    @pl.when(s == 15)
    def _():
        pltpu.sync_copy(ei_hbm.at[:, pl.ds(base + tile_rows, KS)],
                        idx_v.at[:, pl.ds(0, KS)])
        group(0)

    plsc.subcore_barrier()
    ob = c * N_ACC + zb
    pltpu.sync_copy(acc_s.at[pl.ds(zb, ROWS_PER_TILE)],
                    aggp_hbm.at[pl.ds(ob, ROWS_PER_TILE)])
    pltpu.sync_copy(cnt_s.at[pl.ds(zb, ROWS_PER_TILE)],
                    cntp_hbm.at[pl.ds(ob, ROWS_PER_TILE)])


def _sc2_body(h1a_hbm, h1b_hbm, ei_hbm, z32_hbm,
              agg2_hbm,
              idx_v, rows_v, acc_s, gsems, ssem):
    c = lax.axis_index("c")
    s = lax.axis_index("s")
    zb = s * ROWS_PER_TILE
    pltpu.sync_copy(z32_hbm.at[pl.ds(zb, ROWS_PER_TILE)],
                    acc_s.at[pl.ds(zb, ROWS_PER_TILE)])
    plsc.subcore_barrier()

    tile_rows = 390
    base = s * tile_rows
    sup_rows = KS2 * SUP2

    def group_for(table_hbm, q):
        gd = [pltpu.async_copy(table_hbm.at[idx_v.at[0, q * KS2 + j]],
                               rows_v.at[j], gsems.at[j])
              for j in range(KS2)]
        sd = []
        for j in range(KS2):
            gd[j].wait()
            sd.append(pltpu.async_copy(rows_v.at[j],
                                       acc_s.at[idx_v.at[1, q * KS2 + j]],
                                       ssem, add=True))
        for d in sd:
            d.wait()

    def run_for(table_hbm):
        def body(g, _):
            r0 = base + g * sup_rows
            pltpu.sync_copy(ei_hbm.at[:, pl.ds(r0, sup_rows)], idx_v)
            for q in range(SUP2):
                group_for(table_hbm, q)
            return 0

        lax.fori_loop(0, tile_rows // sup_rows, body, 0)

        @pl.when(s == 15)
        def _():
            pltpu.sync_copy(ei_hbm.at[:, pl.ds(base + tile_rows, 2 * KS2)],
                            idx_v.at[:, pl.ds(0, 2 * KS2)])
            group_for(table_hbm, 0)
            group_for(table_hbm, 1)

    @pl.when(c == 0)
    def _():
        run_for(h1a_hbm)

    @pl.when(c == 1)
    def _():
        run_for(h1b_hbm)

    plsc.subcore_barrier()
    ob = c * N_ACC + zb
    pltpu.sync_copy(acc_s.at[pl.ds(zb, ROWS_PER_TILE)],
                    agg2_hbm.at[pl.ds(ob, ROWS_PER_TILE)])


def _dot(a, b):
    return jnp.dot(a, b, precision=lax.Precision.HIGHEST)


def _tc_a_body(agg0_ref, agg1_ref, cnt0_ref, cnt1_ref, x_ref,
               w1al_ref, w1ar_ref, b1a_ref, w1bl_ref, w1br_ref, b1b_ref,
               h1a_ref, h1b_ref):
    agg = agg0_ref[...] + agg1_ref[...]
    cnt = cnt0_ref[...] + cnt1_ref[...]
    inv = 1.0 / jnp.maximum(cnt, 1.0)
    mean = agg * inv
    xb = x_ref[...]
    h1a_ref[...] = jnp.maximum(
        _dot(mean, w1al_ref[...]) + _dot(xb, w1ar_ref[...]) + b1a_ref[...],
        0.0)
    h1b_ref[...] = jnp.maximum(
        _dot(mean, w1bl_ref[...]) + _dot(xb, w1br_ref[...]) + b1b_ref[...],
        0.0)


def _tc_b_body(h1a_ref, h1b_ref, a20_ref, a21_ref, cnt0_ref, cnt1_ref,
               w2la_ref, w2lb_ref, onesk_ref, w2ra_ref, w2rb_ref, b2_ref,
               wm1_ref, bm1_ref, wm2_ref, bm2_ref, out_ref):
    inv = 1.0 / jnp.maximum(cnt0_ref[...] + cnt1_ref[...], 1.0)
    inv32 = _dot(inv, onesk_ref[...])
    h2 = (_dot(a20_ref[...] * inv32, w2la_ref[...])
          + _dot(a21_ref[...] * inv32, w2lb_ref[...])
          + _dot(h1a_ref[...], w2ra_ref[...])
          + _dot(h1b_ref[...], w2rb_ref[...]) + b2_ref[...])
    z = jnp.maximum(_dot(h2, wm1_ref[...]) + bm1_ref[...], 0.0)
    out_ref[...] = _dot(z, wm2_ref[...]) + bm2_ref[...]


_MESH = plsc.VectorSubcoreMesh(core_axis_name="c", subcore_axis_name="s")

_SC_PARAMS = pltpu.CompilerParams(use_tc_tiling_on_sc=False)

_sc1 = pl.kernel(
    _sc1_body,
    out_type=(jax.ShapeDtypeStruct((2 * N_ACC, V_IN), jnp.float32),
              jax.ShapeDtypeStruct((2 * N_ACC, V_IN), jnp.float32)),
    mesh=_MESH,
    compiler_params=_SC_PARAMS,
    scratch_types=[
        pltpu.VMEM((2, KS * SUP1, UN), jnp.int32),
        pltpu.VMEM((KS, UN, V_IN), jnp.float32),
        pltpu.VMEM((UN, V_IN), jnp.float32),
        pltpu.VMEM_SHARED((N_ACC, V_IN), jnp.float32),
        pltpu.VMEM_SHARED((N_ACC, V_IN), jnp.float32),
        pltpu.SemaphoreType.DMA((KS,)),
        pltpu.SemaphoreType.DMA,
    ],
)

_sc2 = pl.kernel(
    _sc2_body,
    out_type=jax.ShapeDtypeStruct((2 * N_ACC, 32), jnp.float32),
    mesh=_MESH,
    compiler_params=_SC_PARAMS,
    scratch_types=[
        pltpu.VMEM((2, KS2 * SUP2, UN), jnp.int32),
        pltpu.VMEM((KS2, UN, 32), jnp.float32),
        pltpu.VMEM_SHARED((N_ACC, 32), jnp.float32),
        pltpu.SemaphoreType.DMA((KS2,)),
        pltpu.SemaphoreType.DMA,
    ],
)

_BR = 3200
_HR = N_NODES // 4


@jax.jit
def kernel(x, edge_index, W1l, b1, W1r, W2l, b2, W2r, Wm1, bm1, Wm2, bm2):
    ei = edge_index.astype(jnp.int32).reshape(2, E_ROWS, UN)

    z16 = jnp.zeros((N_ACC, V_IN), jnp.float32)
    z32 = jnp.zeros((N_ACC, 32), jnp.float32)
    ones = jnp.ones((UN, V_IN), jnp.float32)

    aggp, cnt16p = _sc1(x, ei, z16, ones)
    aggw = aggp.reshape(2 * N_ACC // 4, 4 * V_IN)
    cntw = cnt16p.reshape(2 * N_ACC // 4, 4 * V_IN)
    x4 = x.reshape(_HR, 4 * V_IN)

    i4 = jnp.eye(4, dtype=jnp.float32)
    kr = lambda w: jnp.kron(i4, w)
    t4 = lambda b: jnp.tile(b, 4).reshape(1, -1)

    grid = (4,)
    h1a, h1b = pl.pallas_call(
        _tc_a_body,
        grid=grid,
        in_specs=[
            pl.BlockSpec((_BR, 64), lambda i: (i, 0)),
            pl.BlockSpec((_BR, 64), lambda i: (4 + i, 0)),
            pl.BlockSpec((_BR, 64), lambda i: (i, 0)),
            pl.BlockSpec((_BR, 64), lambda i: (4 + i, 0)),
            pl.BlockSpec((_BR, 64), lambda i: (i, 0)),
            pl.BlockSpec((64, 128), lambda i: (0, 0)),
            pl.BlockSpec((64, 128), lambda i: (0, 0)),
            pl.BlockSpec((1, 128), lambda i: (0, 0)),
            pl.BlockSpec((64, 128), lambda i: (0, 0)),
            pl.BlockSpec((64, 128), lambda i: (0, 0)),
            pl.BlockSpec((1, 128), lambda i: (0, 0)),
        ],
        out_specs=[pl.BlockSpec((_BR, 128), lambda i: (i, 0)),
                   pl.BlockSpec((_BR, 128), lambda i: (i, 0))],
        out_shape=[jax.ShapeDtypeStruct((_HR, 128), jnp.float32),
                   jax.ShapeDtypeStruct((_HR, 128), jnp.float32)],
    )(aggw, aggw, cntw, cntw, x4,
      kr(W1l[:, :32]), kr(W1r[:, :32]), t4(b1[:32]),
      kr(W1l[:, 32:]), kr(W1r[:, 32:]), t4(b1[32:]))

    agg2 = _sc2(h1a.reshape(N_NODES, 32), h1b.reshape(N_NODES, 32), ei, z32)
    agg2w = agg2.reshape(2 * N_ACC // 4, 128)
    onesk = jnp.full((V_IN, 32), 1.0 / V_IN, jnp.float32)

    out = pl.pallas_call(
        _tc_b_body,
        grid=grid,
        in_specs=[
            pl.BlockSpec((_BR, 128), lambda i: (i, 0)),
            pl.BlockSpec((_BR, 128), lambda i: (i, 0)),
            pl.BlockSpec((_BR, 128), lambda i: (i, 0)),
            pl.BlockSpec((_BR, 128), lambda i: (4 + i, 0)),
            pl.BlockSpec((_BR, 64), lambda i: (i, 0)),
            pl.BlockSpec((_BR, 64), lambda i: (4 + i, 0)),
            pl.BlockSpec((128, 256), lambda i: (0, 0)),
            pl.BlockSpec((128, 256), lambda i: (0, 0)),
            pl.BlockSpec((64, 128), lambda i: (0, 0)),
            pl.BlockSpec((128, 256), lambda i: (0, 0)),
            pl.BlockSpec((128, 256), lambda i: (0, 0)),
            pl.BlockSpec((1, 256), lambda i: (0, 0)),
            pl.BlockSpec((256, 256), lambda i: (0, 0)),
            pl.BlockSpec((1, 256), lambda i: (0, 0)),
            pl.BlockSpec((256, 24), lambda i: (0, 0)),
            pl.BlockSpec((1, 24), lambda i: (0, 0)),
        ],
        out_specs=pl.BlockSpec((_BR, 24), lambda i: (i, 0)),
        out_shape=jax.ShapeDtypeStruct((_HR, 24), jnp.float32),
    )(h1a, h1b, agg2w, agg2w, cntw, cntw,
      kr(W2l[:32, :]), kr(W2l[32:, :]), kr(onesk),
      kr(W2r[:32, :]), kr(W2r[32:, :]), t4(b2),
      kr(Wm1), t4(bm1), kr(Wm2), t4(bm2))
    return out.reshape(N_NODES, C_OUT)


# speedup vs baseline: 1.2971x; 1.2971x over previous
import functools

import jax
import jax.numpy as jnp
from jax import lax
from jax.experimental import pallas as pl
from jax.experimental.pallas import tpu as pltpu
from jax.experimental.pallas import tpu_sc as plsc

N_NODES = 50000
N_EDGES = 800000
V_IN = 16
H_DIM = 64
C_OUT = 6

UN = 128
KS = 5
KS2 = 5
SUP1 = 3
SUP2 = 6
E_ROWS = N_EDGES // UN
N_ACC = 51200
ROWS_PER_TILE = N_ACC // 16


def _sc1_body(x_hbm, ei_hbm, z16_hbm, ones_hbm,
              aggp_hbm, cntp_hbm,
              idx_v, rows_v, ones_v, acc_s, cnt_s, gsems, ssem):
    c = lax.axis_index("c")
    s = lax.axis_index("s")
    zb = s * ROWS_PER_TILE
    pltpu.sync_copy(z16_hbm.at[pl.ds(zb, ROWS_PER_TILE)],
                    acc_s.at[pl.ds(zb, ROWS_PER_TILE)])
    pltpu.sync_copy(z16_hbm.at[pl.ds(zb, ROWS_PER_TILE)],
                    cnt_s.at[pl.ds(zb, ROWS_PER_TILE)])
    pltpu.sync_copy(ones_hbm, ones_v)
    plsc.subcore_barrier()

    def group(q):
        gd = [pltpu.async_copy(x_hbm.at[idx_v.at[0, q * KS + j]],
                               rows_v.at[j], gsems.at[j])
              for j in range(KS)]
        sd = []
        for j in range(KS):
            gd[j].wait()
            sd.append(pltpu.async_copy(rows_v.at[j],
                                       acc_s.at[idx_v.at[1, q * KS + j]],
                                       ssem, add=True))
            sd.append(pltpu.async_copy(ones_v,
                                       cnt_s.at[idx_v.at[1, q * KS + j]],
                                       ssem, add=True))
        for d in sd:
            d.wait()

    rows_per_sc = E_ROWS // 2
    tile_rows = 195
    base = c * rows_per_sc + s * tile_rows
    sup_rows = KS * SUP1

    def body(g, _):
        r0 = base + g * sup_rows
        pltpu.sync_copy(ei_hbm.at[:, pl.ds(r0, sup_rows)], idx_v)
        for q in range(SUP1):
            group(q)
        return 0

    lax.fori_loop(0, tile_rows // sup_rows, body, 0)

    @pl.when(s == 15)
    def _():
        pltpu.sync_copy(ei_hbm.at[:, pl.ds(base + tile_rows, KS)],
                        idx_v.at[:, pl.ds(0, KS)])
        group(0)

    plsc.subcore_barrier()
    ob = c * N_ACC + zb
    pltpu.sync_copy(acc_s.at[pl.ds(zb, ROWS_PER_TILE)],
                    aggp_hbm.at[pl.ds(ob, ROWS_PER_TILE)])
    pltpu.sync_copy(cnt_s.at[pl.ds(zb, ROWS_PER_TILE)],
                    cntp_hbm.at[pl.ds(ob, ROWS_PER_TILE)])


def _sc2_body(h1a_hbm, h1b_hbm, ei_hbm, z32_hbm,
              agg2_hbm,
              idx_v, rows_v, acc_s, gsems, ssem):
    c = lax.axis_index("c")
    s = lax.axis_index("s")
    zb = s * ROWS_PER_TILE
    pltpu.sync_copy(z32_hbm.at[pl.ds(zb, ROWS_PER_TILE)],
                    acc_s.at[pl.ds(zb, ROWS_PER_TILE)])
    plsc.subcore_barrier()

    tile_rows = 390
    base = s * tile_rows
    sup_rows = KS2 * SUP2

    def group_for(table_hbm, q):
        gd = [pltpu.async_copy(table_hbm.at[idx_v.at[0, q * KS2 + j]],
                               rows_v.at[j], gsems.at[j])
              for j in range(KS2)]
        sd = []
        for j in range(KS2):
            gd[j].wait()
            sd.append(pltpu.async_copy(rows_v.at[j],
                                       acc_s.at[idx_v.at[1, q * KS2 + j]],
                                       ssem, add=True))
        for d in sd:
            d.wait()

    def run_for(table_hbm):
        def body(g, _):
            r0 = base + g * sup_rows
            pltpu.sync_copy(ei_hbm.at[:, pl.ds(r0, sup_rows)], idx_v)
            for q in range(SUP2):
                group_for(table_hbm, q)
            return 0

        lax.fori_loop(0, tile_rows // sup_rows, body, 0)

        @pl.when(s == 15)
        def _():
            pltpu.sync_copy(ei_hbm.at[:, pl.ds(base + tile_rows, 2 * KS2)],
                            idx_v.at[:, pl.ds(0, 2 * KS2)])
            group_for(table_hbm, 0)
            group_for(table_hbm, 1)

    @pl.when(c == 0)
    def _():
        run_for(h1a_hbm)

    @pl.when(c == 1)
    def _():
        run_for(h1b_hbm)

    plsc.subcore_barrier()
    ob = c * N_ACC + zb
    pltpu.sync_copy(acc_s.at[pl.ds(zb, ROWS_PER_TILE)],
                    agg2_hbm.at[pl.ds(ob, ROWS_PER_TILE)])


def _tc_a_body(agg0_ref, agg1_ref, cnt0_ref, cnt1_ref, x_ref,
               w1al_ref, w1ar_ref, b1a_ref, w1bl_ref, w1br_ref, b1b_ref,
               h1a_ref, h1b_ref):
    agg = agg0_ref[...] + agg1_ref[...]
    cnt = cnt0_ref[...] + cnt1_ref[...]
    inv = 1.0 / jnp.maximum(cnt, 1.0)
    mean = agg * inv
    xb = x_ref[...]
    h1a_ref[...] = jnp.maximum(
        mean @ w1al_ref[...] + xb @ w1ar_ref[...] + b1a_ref[...], 0.0)
    h1b_ref[...] = jnp.maximum(
        mean @ w1bl_ref[...] + xb @ w1br_ref[...] + b1b_ref[...], 0.0)


def _tc_b_body(h1a_ref, h1b_ref, a20_ref, a21_ref, cnt0_ref, cnt1_ref,
               w2la_ref, w2lb_ref, onesk_ref, w2ra_ref, w2rb_ref, b2_ref,
               wm1_ref, bm1_ref, wm2_ref, bm2_ref, out_ref):
    g2 = a20_ref[...] @ w2la_ref[...] + a21_ref[...] @ w2lb_ref[...]
    inv = 1.0 / jnp.maximum(cnt0_ref[...] + cnt1_ref[...], 1.0)
    inv64 = inv @ onesk_ref[...]
    h2 = (g2 * inv64 + h1a_ref[...] @ w2ra_ref[...]
          + h1b_ref[...] @ w2rb_ref[...] + b2_ref[...])
    z = jnp.maximum(h2 @ wm1_ref[...] + bm1_ref[...], 0.0)
    out_ref[...] = z @ wm2_ref[...] + bm2_ref[...]


_MESH = plsc.VectorSubcoreMesh(core_axis_name="c", subcore_axis_name="s")

_SC_PARAMS = pltpu.CompilerParams(use_tc_tiling_on_sc=False)

_sc1 = pl.kernel(
    _sc1_body,
    out_type=(jax.ShapeDtypeStruct((2 * N_ACC, V_IN), jnp.float32),
              jax.ShapeDtypeStruct((2 * N_ACC, V_IN), jnp.float32)),
    mesh=_MESH,
    compiler_params=_SC_PARAMS,
    scratch_types=[
        pltpu.VMEM((2, KS * SUP1, UN), jnp.int32),
        pltpu.VMEM((KS, UN, V_IN), jnp.float32),
        pltpu.VMEM((UN, V_IN), jnp.float32),
        pltpu.VMEM_SHARED((N_ACC, V_IN), jnp.float32),
        pltpu.VMEM_SHARED((N_ACC, V_IN), jnp.float32),
        pltpu.SemaphoreType.DMA((KS,)),
        pltpu.SemaphoreType.DMA,
    ],
)

_sc2 = pl.kernel(
    _sc2_body,
    out_type=jax.ShapeDtypeStruct((2 * N_ACC, 32), jnp.float32),
    mesh=_MESH,
    compiler_params=_SC_PARAMS,
    scratch_types=[
        pltpu.VMEM((2, KS2 * SUP2, UN), jnp.int32),
        pltpu.VMEM((KS2, UN, 32), jnp.float32),
        pltpu.VMEM_SHARED((N_ACC, 32), jnp.float32),
        pltpu.SemaphoreType.DMA((KS2,)),
        pltpu.SemaphoreType.DMA,
    ],
)

_BR = 3200
_HR = N_NODES // 4


@jax.jit
def kernel(x, edge_index, W1l, b1, W1r, W2l, b2, W2r, Wm1, bm1, Wm2, bm2):
    ei = edge_index.astype(jnp.int32).reshape(2, E_ROWS, UN)

    z16 = jnp.zeros((N_ACC, V_IN), jnp.float32)
    z32 = jnp.zeros((N_ACC, 32), jnp.float32)
    ones = jnp.ones((UN, V_IN), jnp.float32)

    aggp, cnt16p = _sc1(x, ei, z16, ones)
    aggw = aggp.reshape(2 * N_ACC // 4, 4 * V_IN)
    cntw = cnt16p.reshape(2 * N_ACC // 4, 4 * V_IN)
    x4 = x.reshape(_HR, 4 * V_IN)

    i4 = jnp.eye(4, dtype=jnp.float32)
    kr = lambda w: jnp.kron(i4, w)
    t4 = lambda b: jnp.tile(b, 4).reshape(1, -1)

    grid = (4,)
    h1a, h1b = pl.pallas_call(
        _tc_a_body,
        grid=grid,
        in_specs=[
            pl.BlockSpec((_BR, 64), lambda i: (i, 0)),
            pl.BlockSpec((_BR, 64), lambda i: (4 + i, 0)),
            pl.BlockSpec((_BR, 64), lambda i: (i, 0)),
            pl.BlockSpec((_BR, 64), lambda i: (4 + i, 0)),
            pl.BlockSpec((_BR, 64), lambda i: (i, 0)),
            pl.BlockSpec((64, 128), lambda i: (0, 0)),
            pl.BlockSpec((64, 128), lambda i: (0, 0)),
            pl.BlockSpec((1, 128), lambda i: (0, 0)),
            pl.BlockSpec((64, 128), lambda i: (0, 0)),
            pl.BlockSpec((64, 128), lambda i: (0, 0)),
            pl.BlockSpec((1, 128), lambda i: (0, 0)),
        ],
        out_specs=[pl.BlockSpec((_BR, 128), lambda i: (i, 0)),
                   pl.BlockSpec((_BR, 128), lambda i: (i, 0))],
        out_shape=[jax.ShapeDtypeStruct((_HR, 128), jnp.float32),
                   jax.ShapeDtypeStruct((_HR, 128), jnp.float32)],
    )(aggw, aggw, cntw, cntw, x4,
      kr(W1l[:, :32]), kr(W1r[:, :32]), t4(b1[:32]),
      kr(W1l[:, 32:]), kr(W1r[:, 32:]), t4(b1[32:]))

    agg2 = _sc2(h1a.reshape(N_NODES, 32), h1b.reshape(N_NODES, 32), ei, z32)
    agg2w = agg2.reshape(2 * N_ACC // 4, 128)
    onesk = jnp.full((V_IN, H_DIM), 1.0 / V_IN, jnp.float32)

    out = pl.pallas_call(
        _tc_b_body,
        grid=grid,
        in_specs=[
            pl.BlockSpec((_BR, 128), lambda i: (i, 0)),
            pl.BlockSpec((_BR, 128), lambda i: (i, 0)),
            pl.BlockSpec((_BR, 128), lambda i: (i, 0)),
            pl.BlockSpec((_BR, 128), lambda i: (4 + i, 0)),
            pl.BlockSpec((_BR, 64), lambda i: (i, 0)),
            pl.BlockSpec((_BR, 64), lambda i: (4 + i, 0)),
            pl.BlockSpec((128, 256), lambda i: (0, 0)),
            pl.BlockSpec((128, 256), lambda i: (0, 0)),
            pl.BlockSpec((64, 256), lambda i: (0, 0)),
            pl.BlockSpec((128, 256), lambda i: (0, 0)),
            pl.BlockSpec((128, 256), lambda i: (0, 0)),
            pl.BlockSpec((1, 256), lambda i: (0, 0)),
            pl.BlockSpec((256, 256), lambda i: (0, 0)),
            pl.BlockSpec((1, 256), lambda i: (0, 0)),
            pl.BlockSpec((256, 24), lambda i: (0, 0)),
            pl.BlockSpec((1, 24), lambda i: (0, 0)),
        ],
        out_specs=pl.BlockSpec((_BR, 24), lambda i: (i, 0)),
        out_shape=jax.ShapeDtypeStruct((_HR, 24), jnp.float32),
    )(h1a, h1b, agg2w, agg2w, cntw, cntw,
      kr(W2l[:32, :]), kr(W2l[32:, :]), kr(onesk),
      kr(W2r[:32, :]), kr(W2r[32:, :]), t4(b2),
      kr(Wm1), t4(bm1), kr(Wm2), t4(bm2))
    return out.reshape(N_NODES, C_OUT)
